# Initial kernel scaffold; baseline (speedup 1.0000x reference)
import jax
import jax.numpy as jnp
from jax.experimental import pallas as pl


def kernel(sim_mat):
    raise NotImplementedError("write your pallas kernel here")


# speedup vs baseline: 1.5798x; 1.5798x over previous
import functools

import jax
import jax.numpy as jnp
from jax.experimental import pallas as pl
from jax.experimental.pallas import tpu as pltpu

_MARGIN = 0.2
_CUT_OFF = 0.5
_D = 512.0
_NONZERO_LOSS_CUTOFF = 1.7
_NEG_BIG = -1e30


def _loss_kernel(sim_ref, g1_ref, g2t_ref, out_ref,
                 colbest_ref, colsim_ref, diag_ref, acc_ref):
    step = pl.program_id(0)
    nsteps = pl.num_programs(0)
    R, B = sim_ref.shape

    sim = sim_ref[...]
    dist = jnp.maximum(jnp.sqrt(2.0 - 2.0 * sim), _CUT_OFF)
    lw = ((2.0 - _D) * jnp.log(dist)
          - (_D - 3.0) / 2.0 * jnp.log(1.0 - 0.25 * (dist * dist)))
    bad = jnp.isinf(lw) | jnp.isnan(lw)

    row_l = jax.lax.broadcasted_iota(jnp.int32, (R, B), 0)
    col = jax.lax.broadcasted_iota(jnp.int32, (R, B), 1)
    offdiag = (row_l + step * R) != col
    in_mask = offdiag & (dist < _NONZERO_LOSS_CUTOFF)
    score_base = jnp.where(in_mask & ~bad, lw, _NEG_BIG)

    diag_entries = jnp.where(offdiag, 0.0, sim)
    diag_blk = jnp.sum(diag_entries, axis=1, keepdims=True)
    diag_row = jnp.sum(diag_entries, axis=0, keepdims=True)

    s1 = score_base + g1_ref[...]
    m1 = jnp.max(s1, axis=1, keepdims=True)
    jstar = jnp.min(jnp.where(s1 == m1, col, B), axis=1, keepdims=True)
    simval1 = jnp.sum(jnp.where(col == jstar, sim, 0.0), axis=1,
                      keepdims=True)
    row_loss = jnp.sum(jnp.maximum(_MARGIN + simval1 - diag_blk, 0.0),
                       keepdims=True)

    s2 = score_base + g2t_ref[...]
    bm = jnp.max(s2, axis=0, keepdims=True)
    rstar = jnp.min(jnp.where(s2 == bm, row_l, R), axis=0, keepdims=True)
    simv = jnp.sum(jnp.where(row_l == rstar, sim, 0.0), axis=0,
                   keepdims=True)

    @pl.when(step == 0)
    def _init():
        acc_ref[...] = jnp.zeros((1, 1), jnp.float32)
        colbest_ref[...] = jnp.full((1, B), -jnp.inf, jnp.float32)
        colsim_ref[...] = jnp.zeros((1, B), jnp.float32)
        diag_ref[...] = jnp.zeros((1, B), jnp.float32)

    acc_ref[...] = acc_ref[...] + row_loss
    diag_ref[...] = diag_ref[...] + diag_row
    better = bm > colbest_ref[...]
    colbest_ref[...] = jnp.where(better, bm, colbest_ref[...])
    colsim_ref[...] = jnp.where(better, simv, colsim_ref[...])

    @pl.when(step == nsteps - 1)
    def _finish():
        col_loss = jnp.sum(jnp.maximum(
            _MARGIN + colsim_ref[...] - diag_ref[...], 0.0), keepdims=True)
        out_ref[...] = acc_ref[...] + col_loss


def kernel(sim_mat):
    B = sim_mat.shape[0]
    key = jax.random.key(42)
    k1, k2 = jax.random.split(key)
    g1 = jax.random.gumbel(k1, (B, B), jnp.float32)
    g2t = jax.random.gumbel(k2, (B, B), jnp.float32).T

    R = 256 if B % 256 == 0 else B
    n = B // R
    out = pl.pallas_call(
        _loss_kernel,
        grid=(n,),
        in_specs=[pl.BlockSpec((R, B), lambda i: (i, 0))] * 3,
        out_specs=pl.BlockSpec((1, 1), lambda i: (0, 0)),
        out_shape=jax.ShapeDtypeStruct((1, 1), jnp.float32),
        scratch_shapes=[
            pltpu.VMEM((1, B), jnp.float32),
            pltpu.VMEM((1, B), jnp.float32),
            pltpu.VMEM((1, B), jnp.float32),
            pltpu.VMEM((1, 1), jnp.float32),
        ],
    )(sim_mat, g1, g2t)
    return out[0, 0]
